# baseline (speedup 1.0000x reference)
import jax
import jax.numpy as jnp
from jax.experimental import pallas as pl
from jax.experimental.pallas import tpu as pltpu

NQ = 4
K = 1024
BB = 2


def _pq_body(x_ref, et_ref, qz_ref, idx_ref, ent_ref, ebf_ref, io_ref):
    nq, cq, k = et_ref.shape
    @pl.when(pl.program_id(0) == 0)
    def _prep():
        for q in range(nq):
            etq = et_ref[q]
            ent_ref[q] = etq / jnp.clip(
                jnp.sqrt(jnp.sum(etq * etq, axis=0, keepdims=True)), 1e-12)
            ebf_ref[q] = jnp.transpose(etq, (1, 0)).astype(jnp.bfloat16)
            rows = jax.lax.broadcasted_iota(jnp.int32, (k, 128), 0)
            cols = jax.lax.broadcasted_iota(jnp.int32, (k, 128), 1)
            io_ref[q] = jnp.where(
                cols == q, (rows // 256) * 256,
                jnp.where(cols == nq + q, rows % 256, 0)
            ).astype(jnp.float32).astype(jnp.bfloat16)

    for i in range(BB):
        xb = x_ref[i]
        acc = None
        for q in range(nq):
            xq = xb[:, q * cq:(q + 1) * cq]
            xn = xq * (1.0 / jnp.clip(
                jnp.sqrt(jnp.sum(xq * xq, axis=1, keepdims=True)), 1e-12))
            dist = jax.lax.dot_general(
                xn, ent_ref[q], (((1,), (0,)), ((), ())),
                preferred_element_type=jnp.float32)
            rowmax = jnp.max(dist, axis=1, keepdims=True)
            one_hot = (dist >= rowmax).astype(jnp.float32).astype(jnp.bfloat16)
            hilo = jax.lax.dot_general(
                one_hot, io_ref[q], (((1,), (0,)), ((), ())),
                preferred_element_type=jnp.float32)
            acc = hilo if acc is None else acc + hilo
            qz_ref[i, :, q * cq:(q + 1) * cq] = jax.lax.dot_general(
                one_hot, ebf_ref[q], (((1,), (0,)), ((), ())),
                preferred_element_type=jnp.float32)
        idx_ref[i] = (acc[:, 0:nq] + acc[:, nq:2 * nq]).astype(jnp.int32)


def kernel(x, embed):
    B, C, H, W = x.shape
    nq, k, cq = embed.shape
    hw = H * W
    xr = jnp.transpose(x, (0, 2, 3, 1)).reshape(B, hw, C)
    et = jnp.transpose(embed, (0, 2, 1))

    qz, idx = pl.pallas_call(
        _pq_body,
        grid=(B // BB,),
        in_specs=[
            pl.BlockSpec((BB, hw, C), lambda b: (b, 0, 0)),
            pl.BlockSpec((nq, cq, k), lambda b: (0, 0, 0)),
        ],
        out_specs=[
            pl.BlockSpec((BB, hw, C), lambda b: (b, 0, 0)),
            pl.BlockSpec((BB, hw, nq), lambda b: (b, 0, 0)),
        ],
        out_shape=[
            jax.ShapeDtypeStruct((B, hw, C), jnp.float32),
            jax.ShapeDtypeStruct((B, hw, nq), jnp.int32),
        ],
        scratch_shapes=[
            pltpu.VMEM((nq, cq, k), jnp.float32),
            pltpu.VMEM((nq, k, cq), jnp.bfloat16),
            pltpu.VMEM((nq, k, 128), jnp.bfloat16),
        ],
        compiler_params=pltpu.CompilerParams(
            dimension_semantics=("arbitrary",)),
    )(xr, et)

    quantized = jnp.transpose(qz.reshape(B, H, W, C), (0, 3, 1, 2))
    encoding = jnp.transpose(idx, (0, 2, 1)).reshape(B, nq * H, W)
    vq_loss = jnp.zeros((1,), dtype=jnp.float32)
    return quantized, encoding, vq_loss


# speedup vs baseline: 1.0611x; 1.0611x over previous
import jax
import jax.numpy as jnp
from jax.experimental import pallas as pl
from jax.experimental.pallas import tpu as pltpu

NQ = 4
K = 1024
BB = 4


def _pq_body(x_ref, et_ref, qz_ref, idx_ref, ent_ref, ebf_ref, io_ref):
    nq, cq, k = et_ref.shape
    @pl.when(pl.program_id(0) == 0)
    def _prep():
        for q in range(nq):
            etq = et_ref[q]
            ent_ref[q] = etq / jnp.clip(
                jnp.sqrt(jnp.sum(etq * etq, axis=0, keepdims=True)), 1e-12)
            ebf_ref[q] = jnp.transpose(etq, (1, 0)).astype(jnp.bfloat16)
            rows = jax.lax.broadcasted_iota(jnp.int32, (k, 128), 0)
            cols = jax.lax.broadcasted_iota(jnp.int32, (k, 128), 1)
            io_ref[q] = jnp.where(
                cols == q, (rows // 256) * 256,
                jnp.where(cols == nq + q, rows % 256, 0)
            ).astype(jnp.float32).astype(jnp.bfloat16)

    for i in range(BB):
        xb = x_ref[i]
        acc = None
        for q in range(nq):
            xq = xb[:, q * cq:(q + 1) * cq]
            xn = xq * (1.0 / jnp.clip(
                jnp.sqrt(jnp.sum(xq * xq, axis=1, keepdims=True)), 1e-12))
            dist = jax.lax.dot_general(
                xn, ent_ref[q], (((1,), (0,)), ((), ())),
                preferred_element_type=jnp.float32)
            rowmax = jnp.max(dist, axis=1, keepdims=True)
            one_hot = (dist >= rowmax).astype(jnp.float32).astype(jnp.bfloat16)
            hilo = jax.lax.dot_general(
                one_hot, io_ref[q], (((1,), (0,)), ((), ())),
                preferred_element_type=jnp.float32)
            acc = hilo if acc is None else acc + hilo
            qz_ref[i, :, q * cq:(q + 1) * cq] = jax.lax.dot_general(
                one_hot, ebf_ref[q], (((1,), (0,)), ((), ())),
                preferred_element_type=jnp.float32)
        idx_ref[i] = (acc[:, 0:nq] + acc[:, nq:2 * nq]).astype(jnp.int32)


def kernel(x, embed):
    B, C, H, W = x.shape
    nq, k, cq = embed.shape
    hw = H * W
    xr = jnp.transpose(x, (0, 2, 3, 1)).reshape(B, hw, C)
    et = jnp.transpose(embed, (0, 2, 1))

    qz, idx = pl.pallas_call(
        _pq_body,
        grid=(B // BB,),
        in_specs=[
            pl.BlockSpec((BB, hw, C), lambda b: (b, 0, 0)),
            pl.BlockSpec((nq, cq, k), lambda b: (0, 0, 0)),
        ],
        out_specs=[
            pl.BlockSpec((BB, hw, C), lambda b: (b, 0, 0)),
            pl.BlockSpec((BB, hw, nq), lambda b: (b, 0, 0)),
        ],
        out_shape=[
            jax.ShapeDtypeStruct((B, hw, C), jnp.float32),
            jax.ShapeDtypeStruct((B, hw, nq), jnp.int32),
        ],
        scratch_shapes=[
            pltpu.VMEM((nq, cq, k), jnp.float32),
            pltpu.VMEM((nq, k, cq), jnp.bfloat16),
            pltpu.VMEM((nq, k, 128), jnp.bfloat16),
        ],
        compiler_params=pltpu.CompilerParams(
            dimension_semantics=("arbitrary",)),
    )(xr, et)

    quantized = jnp.transpose(qz.reshape(B, H, W, C), (0, 3, 1, 2))
    encoding = jnp.transpose(idx, (0, 2, 1)).reshape(B, nq * H, W)
    vq_loss = jnp.zeros((1,), dtype=jnp.float32)
    return quantized, encoding, vq_loss
